# baseline (speedup 1.0000x reference)
import functools
import jax
import jax.numpy as jnp
from jax import lax
from jax.experimental import pallas as pl
from jax.experimental.pallas import tpu as pltpu

V = 128
D = 128
THR = 0.1
TEMP = 0.01
B = 4
L = 64
NE = 1024

ROWS_MAIN = B * V * V
ROWS_ALL = ROWS_MAIN + B * V
RC = 2048
NB = ROWS_MAIN // RC
CPB = (V * V) // RC


def _attn_kernel(x_ref, wq_ref, bq_ref, wk_ref, bk_ref, out_ref, q_s, k_s):
    x = x_ref[0].reshape(L * V, D)
    q = lax.dot_general(x, wq_ref[...], (((1,), (1,)), ((), ()))) + bq_ref[...]
    k = lax.dot_general(x, wk_ref[...], (((1,), (1,)), ((), ()))) + bk_ref[...]
    q_s[...] = q.reshape(L, V, D)
    k_s[...] = k.reshape(L, V, D)

    sqrt_d = jnp.sqrt(jnp.float32(D))

    def body(l, _):
        ql = q_s[l]
        kl = k_s[l]
        s = lax.dot_general(ql, kl, (((1,), (1,)), ((), ())))
        s = s / sqrt_d
        s = s / TEMP
        m1 = jnp.max(s, axis=1, keepdims=True)
        e1 = jnp.exp(s - m1)
        a1 = e1 / jnp.sum(e1, axis=1, keepdims=True)
        s2 = a1 / TEMP
        m2 = jnp.max(s2, axis=0, keepdims=True)
        e2 = jnp.exp(s2 - m2)
        a2 = e2 / jnp.sum(e2, axis=0, keepdims=True)
        a3 = (a2 + a2.T) / 2.0
        a4 = jnp.where(a3 >= THR, a3, jnp.zeros_like(a3))
        out_ref[0, l] = a4
        return 0

    lax.fori_loop(0, L, body, 0, unroll=False)


def _emit_kernel(w_ref, adj_ref, ew_ref, ei_ref):
    p = pl.program_id(0)
    at = w_ref[0].T
    w = at + adj_ref[...]
    r = p * RC + lax.broadcasted_iota(jnp.int32, (RC, 1), 0)
    i_loc = lax.shift_right_logical(r, 7) & (V - 1)
    j_loc = r & (V - 1)
    diag = i_loc == j_loc
    w = jnp.where(diag, 0.0, w)
    is_tail = r >= ROWS_MAIN
    w_out = jnp.where(is_tail, 1.0, w)
    valid = w_out != 0.0
    rowid = lax.shift_right_logical(r, 7)
    colid = lax.shift_left(lax.shift_right_logical(r, 14), 7) | j_loc
    n_tail = r - ROWS_MAIN
    ei0 = jnp.where(is_tail, n_tail, jnp.where(valid, rowid, -1))
    ei1 = jnp.where(is_tail, n_tail, jnp.where(valid, colid, -1))
    ew_ref[...] = w_out
    ei_ref[0] = ei0
    ei_ref[1] = ei1


def _build_adj_flat(edge_index, edge_weight):
    e0 = edge_index[0]
    e1 = edge_index[1]
    idx = e0 * V + lax.rem(e1, V)
    return jnp.zeros((ROWS_MAIN,), jnp.float32).at[idx].add(edge_weight)


@jax.jit
def kernel(hidden_states, edge_index, edge_weight, Wq, bq, Wk, bk):
    adj_flat = _build_adj_flat(edge_index, edge_weight)

    h_t = jnp.transpose(hidden_states.reshape(B, V, L, D), (0, 2, 1, 3))

    a4 = pl.pallas_call(
        _attn_kernel,
        grid=(B,),
        in_specs=[
            pl.BlockSpec((1, L, V, D), lambda b: (b, 0, 0, 0)),
            pl.BlockSpec((D, D), lambda b: (0, 0)),
            pl.BlockSpec((1, D), lambda b: (0, 0)),
            pl.BlockSpec((D, D), lambda b: (0, 0)),
            pl.BlockSpec((1, D), lambda b: (0, 0)),
        ],
        out_specs=pl.BlockSpec((1, L, V, V), lambda b: (b, 0, 0, 0)),
        out_shape=jax.ShapeDtypeStruct((B, L, V, V), jnp.float32),
        scratch_shapes=[
            pltpu.VMEM((L, V, D), jnp.float32),
            pltpu.VMEM((L, V, D), jnp.float32),
        ],
    )(h_t, Wq, bq.reshape(1, D), Wk, bk.reshape(1, D))

    w2 = a4.reshape(B, L, V * V)
    adj2 = adj_flat.reshape(ROWS_MAIN, 1)

    ew, ei = pl.pallas_call(
        _emit_kernel,
        grid=(NB + 1,),
        in_specs=[
            pl.BlockSpec(
                (1, L, RC),
                lambda p: (jnp.minimum(p // CPB, B - 1), 0,
                           jnp.where(p >= NB, CPB - 1, lax.rem(p, CPB))),
            ),
            pl.BlockSpec((RC, 1), lambda p: (jnp.minimum(p, NB - 1), 0)),
        ],
        out_specs=[
            pl.BlockSpec((RC, L), lambda p: (p, 0)),
            pl.BlockSpec((2, RC, L), lambda p: (0, p, 0)),
        ],
        out_shape=[
            jax.ShapeDtypeStruct((ROWS_ALL, L), jnp.float32),
            jax.ShapeDtypeStruct((2, ROWS_ALL, L), jnp.int32),
        ],
    )(w2, adj2)

    return ei, ew


# speedup vs baseline: 1.1059x; 1.1059x over previous
import functools
import jax
import jax.numpy as jnp
from jax import lax
from jax.experimental import pallas as pl
from jax.experimental.pallas import tpu as pltpu
from jax.experimental.pallas import tpu_sc as plsc

V = 128
D = 128
THR = 0.1
TEMP = 0.01
B = 4
L = 64
NE = 1024

ROWS_MAIN = B * V * V
ROWS_ALL = ROWS_MAIN + B * V
RC = 2048
NB = ROWS_MAIN // RC
CPB = (V * V) // RC


def _attn_kernel(x_ref, wq_ref, bq_ref, wk_ref, bk_ref, out_ref, q_s, k_s):
    x = x_ref[0].reshape(L * V, D)
    q = lax.dot_general(x, wq_ref[...], (((1,), (1,)), ((), ()))) + bq_ref[...]
    k = lax.dot_general(x, wk_ref[...], (((1,), (1,)), ((), ()))) + bk_ref[...]
    q_s[...] = q.reshape(L, V, D)
    k_s[...] = k.reshape(L, V, D)

    sqrt_d = jnp.sqrt(jnp.float32(D))

    def body(l, _):
        ql = q_s[l]
        kl = k_s[l]
        s = lax.dot_general(ql, kl, (((1,), (1,)), ((), ())))
        s = s / sqrt_d
        s = s / TEMP
        m1 = jnp.max(s, axis=1, keepdims=True)
        e1 = jnp.exp(s - m1)
        a1 = e1 / jnp.sum(e1, axis=1, keepdims=True)
        s2 = a1 / TEMP
        m2 = jnp.max(s2, axis=0, keepdims=True)
        e2 = jnp.exp(s2 - m2)
        a2 = e2 / jnp.sum(e2, axis=0, keepdims=True)
        a3 = (a2 + a2.T) / 2.0
        a4 = jnp.where(a3 >= THR, a3, jnp.zeros_like(a3))
        out_ref[0, l] = a4
        return 0

    lax.fori_loop(0, L, body, 0, unroll=False)


def _emit_kernel(w_ref, adj_ref, ew_ref, ei_ref):
    p = pl.program_id(0)
    at = w_ref[0].T
    w = at + adj_ref[...]
    r = p * RC + lax.broadcasted_iota(jnp.int32, (RC, 1), 0)
    i_loc = lax.shift_right_logical(r, 7) & (V - 1)
    j_loc = r & (V - 1)
    diag = i_loc == j_loc
    w = jnp.where(diag, 0.0, w)
    is_tail = r >= ROWS_MAIN
    w_out = jnp.where(is_tail, 1.0, w)
    valid = w_out != 0.0
    rowid = lax.shift_right_logical(r, 7)
    colid = lax.shift_left(lax.shift_right_logical(r, 14), 7) | j_loc
    n_tail = r - ROWS_MAIN
    ei0 = jnp.where(is_tail, n_tail, jnp.where(valid, rowid, -1))
    ei1 = jnp.where(is_tail, n_tail, jnp.where(valid, colid, -1))
    ew_ref[...] = w_out
    ei_ref[0] = ei0
    ei_ref[1] = ei1


E_TOT = B * NE
NCORE = 2
NSUB = 16
EPS = E_TOT // NSUB
HALF = ROWS_MAIN // NCORE
CPW = HALF // NSUB
ACC_N = HALF + 16


def _adj_sc_kernel(e0_hbm, e1_hbm, w_hbm, out_hbm,
                   e0_v, e1_v, w_v, idx_v, stage_v, acc_sh):
    c = lax.axis_index("c")
    s = lax.axis_index("s")

    def zbody(t, _):
        stage_v[pl.ds(t * 16, 16)] = jnp.zeros((16,), jnp.float32)
        return 0
    lax.fori_loop(0, CPW // 16, zbody, 0, unroll=False)
    pltpu.sync_copy(stage_v, acc_sh.at[pl.ds(s * CPW, CPW)])

    @pl.when(s == 0)
    def _():
        pltpu.sync_copy(stage_v.at[pl.ds(0, 16)], acc_sh.at[pl.ds(HALF, 16)])

    pltpu.sync_copy(e0_hbm.at[pl.ds(s * EPS, EPS)], e0_v)
    pltpu.sync_copy(e1_hbm.at[pl.ds(s * EPS, EPS)], e1_v)
    pltpu.sync_copy(w_hbm.at[pl.ds(s * EPS, EPS)], w_v)

    base = c * HALF
    for t in range(EPS // 16):
        sl = pl.ds(t * 16, 16)
        e0 = e0_v[sl]
        e1 = e1_v[sl]
        idx = lax.shift_left(e0, 7) | (e1 & (V - 1))
        loc = idx - base
        inr = (loc >= 0) & (loc < HALF)
        idx_v[sl] = jnp.where(inr, loc, HALF)

    plsc.subcore_barrier()
    pltpu.sync_copy(w_v, acc_sh.at[idx_v], add=True)
    plsc.subcore_barrier()

    pltpu.sync_copy(acc_sh.at[pl.ds(s * CPW, CPW)],
                    out_hbm.at[pl.ds(base + s * CPW, CPW)])


def _build_adj_flat(edge_index, edge_weight):
    f = functools.partial(
        pl.kernel,
        mesh=plsc.VectorSubcoreMesh(core_axis_name="c", subcore_axis_name="s"),
        out_type=jax.ShapeDtypeStruct((ROWS_MAIN,), jnp.float32),
        scratch_types=[
            pltpu.VMEM((EPS,), jnp.int32),
            pltpu.VMEM((EPS,), jnp.int32),
            pltpu.VMEM((EPS,), jnp.float32),
            pltpu.VMEM((EPS,), jnp.int32),
            pltpu.VMEM((CPW,), jnp.float32),
            pltpu.VMEM_SHARED((ACC_N,), jnp.float32),
        ],
    )(_adj_sc_kernel)
    return f(edge_index[0], edge_index[1], edge_weight)


@jax.jit
def kernel(hidden_states, edge_index, edge_weight, Wq, bq, Wk, bk):
    adj_flat = _build_adj_flat(edge_index, edge_weight)

    h_t = jnp.transpose(hidden_states.reshape(B, V, L, D), (0, 2, 1, 3))

    a4 = pl.pallas_call(
        _attn_kernel,
        grid=(B,),
        in_specs=[
            pl.BlockSpec((1, L, V, D), lambda b: (b, 0, 0, 0)),
            pl.BlockSpec((D, D), lambda b: (0, 0)),
            pl.BlockSpec((1, D), lambda b: (0, 0)),
            pl.BlockSpec((D, D), lambda b: (0, 0)),
            pl.BlockSpec((1, D), lambda b: (0, 0)),
        ],
        out_specs=pl.BlockSpec((1, L, V, V), lambda b: (b, 0, 0, 0)),
        out_shape=jax.ShapeDtypeStruct((B, L, V, V), jnp.float32),
        scratch_shapes=[
            pltpu.VMEM((L, V, D), jnp.float32),
            pltpu.VMEM((L, V, D), jnp.float32),
        ],
    )(h_t, Wq, bq.reshape(1, D), Wk, bk.reshape(1, D))

    w2 = a4.reshape(B, L, V * V)
    adj2 = adj_flat.reshape(ROWS_MAIN, 1)

    ew, ei = pl.pallas_call(
        _emit_kernel,
        grid=(NB + 1,),
        in_specs=[
            pl.BlockSpec(
                (1, L, RC),
                lambda p: (jnp.minimum(p // CPB, B - 1), 0,
                           jnp.where(p >= NB, CPB - 1, lax.rem(p, CPB))),
            ),
            pl.BlockSpec((RC, 1), lambda p: (jnp.minimum(p, NB - 1), 0)),
        ],
        out_specs=[
            pl.BlockSpec((RC, L), lambda p: (p, 0)),
            pl.BlockSpec((2, RC, L), lambda p: (0, p, 0)),
        ],
        out_shape=[
            jax.ShapeDtypeStruct((ROWS_ALL, L), jnp.float32),
            jax.ShapeDtypeStruct((2, ROWS_ALL, L), jnp.int32),
        ],
    )(w2, adj2)

    return ei, ew
